# baseline (device time: 66366 ns/iter reference)
import jax
import jax.numpy as jnp
from jax import lax
from jax.experimental import pallas as pl
from jax.experimental.pallas import tpu as pltpu


def kernel(Q, K, V):
    b, sq, h, d = Q.shape
    scale = d ** -0.5

    Qd = jnp.transpose(Q * scale, (0, 2, 3, 1)).astype(jnp.bfloat16)
    Kd = jnp.transpose(K, (0, 2, 3, 1)).astype(jnp.bfloat16)
    Vd = jnp.transpose(V, (0, 2, 3, 1)).astype(jnp.bfloat16)

    dn_s = (((1,), (1,)), ((0,), (0,)))
    dn_o = (((2,), (1,)), ((0,), (0,)))

    def body(kany, vany, qd, kd, vd, out_ref, krem, vrem, o0, mstats,
             ksend, krecv, vsend, vrecv):
        p = pl.program_id(0)
        bi = pl.program_id(1)
        my_x = lax.axis_index("x")
        my_y = lax.axis_index("y")
        my_z = lax.axis_index("z")
        partner = (1 - my_x, my_y, my_z)

        def chunk_rdma(i):
            rk = pltpu.make_async_remote_copy(
                src_ref=kany.at[i], dst_ref=krem.at[i],
                send_sem=ksend.at[i], recv_sem=krecv.at[i],
                device_id=partner, device_id_type=pl.DeviceIdType.MESH)
            rv = pltpu.make_async_remote_copy(
                src_ref=vany.at[i], dst_ref=vrem.at[i],
                send_sem=vsend.at[i], recv_sem=vrecv.at[i],
                device_id=partner, device_id_type=pl.DeviceIdType.MESH)
            return rk, rv

        @pl.when(jnp.logical_and(p == 0, bi == 0))
        def _():
            barrier = pltpu.get_barrier_semaphore()
            pl.semaphore_signal(barrier, inc=1, device_id=partner,
                                device_id_type=pl.DeviceIdType.MESH)
            pl.semaphore_wait(barrier, 1)
            for i in range(b):
                rk, rv = chunk_rdma(i)
                rk.start()
                rv.start()

        @pl.when(p == 1)
        def _():
            rk, rv = chunk_rdma(bi)
            rk.wait()
            rv.wait()

        q = qd[bi]

        @pl.when(p == 0)
        def _():
            s0 = lax.dot_general(kd[0], q, dn_s,
                                 preferred_element_type=jnp.float32)
            p0 = jnp.exp(s0)
            l0 = jnp.sum(p0, axis=1, keepdims=True)
            o0[bi] = lax.dot_general(vd[0], p0.astype(jnp.bfloat16), dn_o,
                                     preferred_element_type=jnp.float32)
            mstats[bi, :, 0:1, :] = l0

        @pl.when(p == 1)
        def _():
            s1 = lax.dot_general(krem[bi], q, dn_s,
                                 preferred_element_type=jnp.float32)
            p1 = jnp.exp(s1)
            l1 = jnp.sum(p1, axis=1, keepdims=True)
            l0 = mstats[bi, :, 0:1, :]
            o1 = lax.dot_general(vrem[bi], p1.astype(jnp.bfloat16), dn_o,
                                 preferred_element_type=jnp.float32)
            out_ref[bi] = ((o0[bi] + o1) / (l0 + l1)).astype(jnp.bfloat16)

    out_t = pl.pallas_call(
        body,
        grid=(2, b),
        in_specs=[
            pl.BlockSpec(memory_space=pl.ANY),
            pl.BlockSpec(memory_space=pl.ANY),
            pl.BlockSpec(memory_space=pltpu.VMEM),
            pl.BlockSpec((1, h, d, sq), lambda p, bi: (bi, 0, 0, 0)),
            pl.BlockSpec((1, h, d, sq), lambda p, bi: (bi, 0, 0, 0)),
        ],
        out_specs=pl.BlockSpec(memory_space=pltpu.VMEM),
        out_shape=jax.ShapeDtypeStruct((b, h, d, sq), jnp.bfloat16),
        scratch_shapes=[
            pltpu.VMEM((b, h, d, sq), jnp.bfloat16),
            pltpu.VMEM((b, h, d, sq), jnp.bfloat16),
            pltpu.VMEM((b, h, d, sq), jnp.float32),
            pltpu.VMEM((b, h, 1, sq), jnp.float32),
            pltpu.SemaphoreType.DMA((b,)),
            pltpu.SemaphoreType.DMA((b,)),
            pltpu.SemaphoreType.DMA((b,)),
            pltpu.SemaphoreType.DMA((b,)),
        ],
        compiler_params=pltpu.CompilerParams(
            collective_id=0,
            vmem_limit_bytes=96 * 1024 * 1024,
        ),
    )(Kd, Vd, Qd, Kd, Vd)

    return jnp.transpose(out_t, (0, 3, 1, 2))
